# Initial kernel scaffold; baseline (speedup 1.0000x reference)
import jax
import jax.numpy as jnp
from jax.experimental import pallas as pl


def kernel(video_src, audio_src, video_pos, audio_pos, video_reference_points, audio_reference_points, video_temporal_shapes, video_level_start_index, audio_temporal_shapes, audio_level_start_index, video_mask_flatten, audio_mask_flatten, params):
    raise NotImplementedError("write your pallas kernel here")


# speedup vs baseline: 22.5364x; 22.5364x over previous
import functools
import numpy as np
import jax
import jax.numpy as jnp
from jax import lax
from jax.experimental import pallas as pl
from jax.experimental.pallas import tpu as pltpu
from jax.experimental.pallas import tpu_sc as plsc

_D = 256
_M = 8
_L = 4
_P = 4
_DH = 32
_DF = 1024
_VID = (8192, 4096, 2048, 1024)
_AUD = (4096, 2048, 1024, 512)
_NW = 32
_CH = 4
_BQ = 512


def _value_body(src_ref, maskf_ref, w_ref, b_ref, out_ref):
    x = src_ref[0]
    v = jnp.dot(x, w_ref[...], preferred_element_type=jnp.float32) + b_ref[...]
    out_ref[0] = v * maskf_ref[0]


def _value_proj(src, maskf, w_t, b):
    n, lin, _ = src.shape
    return pl.pallas_call(
        _value_body,
        grid=(n, lin // _BQ),
        in_specs=[
            pl.BlockSpec((1, _BQ, _D), lambda i, j: (i, j, 0)),
            pl.BlockSpec((1, _BQ, 1), lambda i, j: (i, j, 0)),
            pl.BlockSpec((_D, _D), lambda i, j: (0, 0)),
            pl.BlockSpec((1, _D), lambda i, j: (0, 0)),
        ],
        out_specs=pl.BlockSpec((1, _BQ, _D), lambda i, j: (i, j, 0)),
        out_shape=jax.ShapeDtypeStruct((n, lin, _D), jnp.float32),
    )(src, maskf, w_t, b)


def _samp_body(q_ref, refe_ref, w_ref, b_ref, tcol_ref, ibase_ref,
               loc_ref, aw_ref, idx_ref, wa_ref, wb_ref, *, lin):
    n = pl.program_id(0)
    q = q_ref[0]
    so_aw = jnp.dot(q, w_ref[...], preferred_element_type=jnp.float32) + b_ref[...]
    so = so_aw[:, :128]
    awl = so_aw[:, 128:]
    ri = lax.broadcasted_iota(jnp.int32, (128, 128), 0) // 16
    ci = lax.broadcasted_iota(jnp.int32, (128, 128), 1) // 16
    seg = (ri == ci).astype(jnp.float32)
    e = jnp.exp(awl)
    aw = e / jnp.dot(e, seg, preferred_element_type=jnp.float32)
    tcol = tcol_ref[...]
    loc = refe_ref[0] + so / tcol
    x = loc * tcol - 0.5
    x0f = jnp.floor(x)
    w1 = x - x0f
    t0 = (x0f >= 0.0) & (x0f <= tcol - 1.0)
    t1 = (x0f >= -1.0) & (x0f <= tcol - 2.0)
    wa = aw * jnp.where(t0, 1.0 - w1, jnp.where(t1, w1, 0.0))
    wb = aw * jnp.where(t0 & t1, w1, 0.0)
    r = jnp.clip(x0f, 0.0, tcol - 1.0).astype(jnp.int32)
    idx = r + ibase_ref[...] + n * (_M * lin)
    loc_ref[0] = loc
    aw_ref[0] = aw
    idx_ref[0] = idx
    wa_ref[0] = wa
    wb_ref[0] = wb


def _samp(query, refe, cat_w, cat_b, tcol, ibase, lin):
    n, lq, _ = query.shape
    grid = (n, lq // _BQ)
    blk = pl.BlockSpec((1, _BQ, 128), lambda i, j: (i, j, 0))
    out_shapes = [jax.ShapeDtypeStruct((n, lq, 128), jnp.float32)] * 2 + \
                 [jax.ShapeDtypeStruct((n, lq, 128), jnp.int32)] + \
                 [jax.ShapeDtypeStruct((n, lq, 128), jnp.float32)] * 2
    return pl.pallas_call(
        functools.partial(_samp_body, lin=lin),
        grid=grid,
        in_specs=[
            pl.BlockSpec((1, _BQ, _D), lambda i, j: (i, j, 0)),
            pl.BlockSpec((1, _BQ, 128), lambda i, j: (i, j, 0)),
            pl.BlockSpec((_D, _D), lambda i, j: (0, 0)),
            pl.BlockSpec((1, _D), lambda i, j: (0, 0)),
            pl.BlockSpec((1, 128), lambda i, j: (0, 0)),
            pl.BlockSpec((1, 128), lambda i, j: (0, 0)),
        ],
        out_specs=[blk] * 5,
        out_shape=out_shapes,
    )(query, refe, cat_w, cat_b, tcol, ibase)


def _outln_body(acc_ref, src_ref, w_ref, b_ref, g_ref, bb_ref, o_ref):
    a = acc_ref[0]
    y = jnp.dot(a, w_ref[...], preferred_element_type=jnp.float32) + b_ref[...]
    x = src_ref[0] + y
    mu = jnp.mean(x, -1, keepdims=True)
    var = jnp.mean((x - mu) ** 2, -1, keepdims=True)
    o_ref[0] = (x - mu) / jnp.sqrt(var + 1e-5) * g_ref[...] + bb_ref[...]


def _outln(acc, src, w_t, b, g, bb):
    n, lq, _ = acc.shape
    blk = pl.BlockSpec((1, _BQ, _D), lambda i, j: (i, j, 0))
    vec = pl.BlockSpec((1, _D), lambda i, j: (0, 0))
    return pl.pallas_call(
        _outln_body,
        grid=(n, lq // _BQ),
        in_specs=[blk, blk, pl.BlockSpec((_D, _D), lambda i, j: (0, 0)),
                  vec, vec, vec],
        out_specs=blk,
        out_shape=jax.ShapeDtypeStruct((n, lq, _D), jnp.float32),
    )(acc, src, w_t, b, g, bb)


def _outffn_body(acc_ref, ow_ref, ob_ref, w1_ref, b1_ref, w2_ref, b2_ref,
                 g_ref, bb_ref, o_ref):
    a = acc_ref[0]
    x = jnp.dot(a, ow_ref[...], preferred_element_type=jnp.float32) + ob_ref[...]
    h = jnp.maximum(
        jnp.dot(x, w1_ref[...], preferred_element_type=jnp.float32) + b1_ref[...],
        0.0)
    y = jnp.dot(h, w2_ref[...], preferred_element_type=jnp.float32) + b2_ref[...]
    x = x + y
    mu = jnp.mean(x, -1, keepdims=True)
    var = jnp.mean((x - mu) ** 2, -1, keepdims=True)
    o_ref[0] = (x - mu) / jnp.sqrt(var + 1e-5) * g_ref[...] + bb_ref[...]


def _outffn(acc, ow_t, ob, w1_t, b1, w2_t, b2, g, bb):
    n, lq, _ = acc.shape
    blk = pl.BlockSpec((1, _BQ, _D), lambda i, j: (i, j, 0))
    vec = pl.BlockSpec((1, _D), lambda i, j: (0, 0))
    return pl.pallas_call(
        _outffn_body,
        grid=(n, lq // _BQ),
        in_specs=[blk,
                  pl.BlockSpec((_D, _D), lambda i, j: (0, 0)), vec,
                  pl.BlockSpec((_D, _DF), lambda i, j: (0, 0)),
                  pl.BlockSpec((1, _DF), lambda i, j: (0, 0)),
                  pl.BlockSpec((_DF, _D), lambda i, j: (0, 0)), vec,
                  vec, vec],
        out_specs=blk,
        out_shape=jax.ShapeDtypeStruct((n, lq, _D), jnp.float32),
    )(acc, ow_t, ob, w1_t, b1, w2_t, b2, g, bb)


_GTR_DNUMS = lax.GatherDimensionNumbers(
    offset_dims=(), collapsed_slice_dims=(0,), start_index_map=(0,))


def _bcast(vec, j):
    idx = jnp.full((16, 1), j, jnp.int32)
    return lax.gather(vec, idx, _GTR_DNUMS, slice_sizes=(1,),
                      mode=lax.GatherScatterMode.PROMISE_IN_BOUNDS)


def _sc_attend(table, idxm, wgtm):
    nq = idxm.shape[0]
    rows_w = nq // _NW
    nch = rows_w // _CH
    mesh = plsc.VectorSubcoreMesh(core_axis_name="c", subcore_axis_name="s")

    @functools.partial(
        pl.kernel,
        out_type=jax.ShapeDtypeStruct((nq * 256,), jnp.float32),
        mesh=mesh,
        scratch_types=[
            pltpu.VMEM((_CH, 128), jnp.int32),
            pltpu.VMEM((_CH * 256,), jnp.float32),
            pltpu.VMEM((_CH * 128, 64), jnp.float32),
            pltpu.VMEM((_CH * 256,), jnp.float32),
            pltpu.SemaphoreType.DMA,
        ],
        compiler_params=pltpu.CompilerParams(use_tc_tiling_on_sc=False),
    )
    def k(table_h, idx_h, wgt_h, out_h, idx_v, wgt_v, gath_v, out_v, sem):
        wid = lax.axis_index("s") * 2 + lax.axis_index("c")
        base = wid * rows_w

        def chunk(ci, carry):
            row0 = base + ci * _CH
            pltpu.sync_copy(idx_h.at[pl.ds(row0, _CH)], idx_v)
            pltpu.sync_copy(
                wgt_h.at[pl.ds(row0 * 256, _CH * 256)], wgt_v)
            cps = []
            for r in range(_CH):
                cp = pltpu.make_async_copy(
                    table_h.at[idx_v.at[r]],
                    gath_v.at[pl.ds(r * 128, 128)], sem)
                cp.start()
                cps.append(cp)
            for cp in cps:
                cp.wait()

            def qrow(r, c2):
                for m in range(_M):
                    gb = r * 128 + m * 16
                    wbase = r * 256 + m * 32
                    wv0 = wgt_v[pl.ds(wbase, 16)]
                    wv1 = wgt_v[pl.ds(wbase + 16, 16)]
                    a0 = jnp.zeros((16,), jnp.float32)
                    a1 = jnp.zeros((16,), jnp.float32)
                    for j in range(16):
                        g = gb + j
                        wv = wv0 if j < 8 else wv1
                        jj = (j % 8) * 2
                        wa = _bcast(wv, jj)
                        wb = _bcast(wv, jj + 1)
                        a0 = a0 + wa * gath_v[g, pl.ds(0, 16)] + wb * gath_v[g, pl.ds(32, 16)]
                        a1 = a1 + wa * gath_v[g, pl.ds(16, 16)] + wb * gath_v[g, pl.ds(48, 16)]
                    out_v[pl.ds(wbase, 16)] = a0
                    out_v[pl.ds(wbase + 16, 16)] = a1
                return c2

            lax.fori_loop(0, _CH, qrow, 0)
            pltpu.sync_copy(
                out_v, out_h.at[pl.ds(row0 * 256, _CH * 256)])
            return carry

        lax.fori_loop(0, nch, chunk, 0)

    return k(table, idxm, wgtm)


def _make_table(value, n, lin):
    flat = value.reshape(n, lin, _M, _DH).transpose(0, 2, 1, 3).reshape(n * _M * lin, _DH)
    flat_pad = jnp.concatenate([flat, jnp.zeros((1, _DH), flat.dtype)], 0)
    return jnp.concatenate([flat_pad[:-1], flat_pad[1:]], 1)


def _expand_ref(refpts, n, lq):
    r = refpts[:, :, :, 0]
    r = jnp.repeat(r, _P, axis=2)
    return jnp.tile(r, (1, 1, _M))


def _col_consts(shapes, lin):
    t = np.zeros((128,), np.float32)
    ib = np.zeros((128,), np.int32)
    starts = np.concatenate([[0], np.cumsum(shapes)[:-1]]).astype(np.int64)
    for c in range(128):
        m = c // 16
        l = (c // 4) % 4
        t[c] = shapes[l]
        ib[c] = m * lin + starts[l]
    return jnp.asarray(t).reshape(1, 128), jnp.asarray(ib).reshape(1, 128)


def kernel(video_src, audio_src, video_pos, audio_pos, video_reference_points,
           audio_reference_points, video_temporal_shapes, video_level_start_index,
           audio_temporal_shapes, audio_level_start_index, video_mask_flatten,
           audio_mask_flatten, params):
    pa = params['attn']
    n, lv, _ = video_src.shape
    la = audio_src.shape[1]

    vw_t = pa['value_w'].T
    vb = pa['value_b'].reshape(1, _D)
    cat_w = jnp.concatenate([pa['so_w'], pa['aw_w']], 0).T
    cat_b = jnp.concatenate([pa['so_b'], pa['aw_b']], 0).reshape(1, _D)
    ow_t = pa['out_w'].T
    ob = pa['out_b'].reshape(1, _D)
    g1 = params['norm1_g'].reshape(1, _D)
    b1 = params['norm1_b'].reshape(1, _D)
    w1_t = params['lin1_w'].T
    bb1 = params['lin1_b'].reshape(1, _DF)
    w2_t = params['lin2_w'].T
    bb2 = params['lin2_b'].reshape(1, _D)
    g2 = params['norm2_g'].reshape(1, _D)
    b2 = params['norm2_b'].reshape(1, _D)

    vmaskf = (1.0 - video_mask_flatten.astype(jnp.float32)).reshape(n, lv, 1)
    amaskf = (1.0 - audio_mask_flatten.astype(jnp.float32)).reshape(n, la, 1)
    vref_e = _expand_ref(video_reference_points, n, lv)
    aref_e = _expand_ref(audio_reference_points, n, la)
    vtcol, vibase = _col_consts(_VID, lv)
    atcol, aibase = _col_consts(_AUD, la)

    def attn(query, refe, val_src, maskf, tcol, ibase, lin):
        lq = query.shape[1]
        value = _value_proj(val_src, maskf, vw_t, vb)
        table = _make_table(value, n, lin)
        loc, aw, idxm, wa, wb = _samp(query, refe, cat_w, cat_b, tcol, ibase, lin)
        wgt = jnp.stack([wa, wb], -1).reshape(n * lq * 256)
        acc = _sc_attend(table, idxm.reshape(n * lq, 128), wgt)
        return acc.reshape(n, lq, _D), loc, aw

    q1 = video_src + video_pos
    acc1, _, _ = attn(q1, vref_e, video_src, vmaskf, vtcol, vibase, lv)
    vs = _outln(acc1, video_src, ow_t, ob, g1, b1)

    q2 = audio_src + audio_pos
    acc2, _, _ = attn(q2, aref_e, audio_src, amaskf, atcol, aibase, la)
    au = _outln(acc2, audio_src, ow_t, ob, g1, b1)

    acc3, a_loc, a_w = attn(au, aref_e, vs, vmaskf, vtcol, vibase, lv)
    visual_attended_audio = _outffn(acc3, ow_t, ob, w1_t, bb1, w2_t, bb2, g2, b2)

    acc4, v_loc, v_w = attn(vs, vref_e, au, amaskf, atcol, aibase, la)
    audio_attended_visual = _outffn(acc4, ow_t, ob, w1_t, bb1, w2_t, bb2, g2, b2)

    v_loc = v_loc.reshape(n, lv, _M, _L, _P)
    v_w = v_w.reshape(n, lv, _M, _L, _P)
    a_loc = a_loc.reshape(n, la, _M, _L, _P)
    a_w = a_w.reshape(n, la, _M, _L, _P)
    return (audio_attended_visual, visual_attended_audio, v_loc, v_w, a_loc, a_w)
